# baseline (speedup 1.0000x reference)
import functools

import jax
import jax.numpy as jnp
from jax import lax
from jax.experimental import pallas as pl
from jax.experimental.pallas import tpu as pltpu
from jax.experimental.pallas import tpu_sc as plsc

N = 4096
E = 16384
F_IN = 11
DIM = 64
EDGE_DIM = 6
B = 256
OUT_CLASSES = 12

NW = 32
EPW = E // NW
CHUNK = 128
NCH = EPW // CHUNK
PADW = 128
RPW = N // 16


def _sigmoid(v):
    return 1.0 / (1.0 + jnp.exp(-v))


def _prologue_body(x_ref, eat_ref, w0_ref, b0_ref, we1t_ref, be1_ref,
                   out_ref, eht_ref):
    out_ref[:, 0:DIM] = jax.nn.relu(
        jnp.dot(x_ref[...], w0_ref[...], preferred_element_type=jnp.float32)
        + b0_ref[...])
    out_ref[:, DIM:PADW] = jnp.zeros((N, PADW - DIM), jnp.float32)
    eht = jax.nn.relu(
        jnp.dot(we1t_ref[...], eat_ref[...], preferred_element_type=jnp.float32)
        + be1_ref[...])
    eht_ref[...] = eht.astype(jnp.bfloat16)


MBLK = 1024
NCK = 4
ICK = DIM // NCK


def _msg_body(eht_ref, sf_ref, we2t_ref, be2_ref, out_ref, *c_refs):
    sf = sf_ref[:, 0:DIM]
    sft = sf.T.astype(jnp.bfloat16)
    eht = eht_ref[...]
    acct = jnp.zeros((DIM, MBLK), jnp.float32)
    for c in range(NCK):
        cr = c_refs[c]
        for k in range(ICK):
            i = c * ICK + k
            cr[k * 128:(k + 1) * 128, :] = eht * sft[i:i + 1, :]
        acct = acct + jnp.dot(
            we2t_ref[:, c * ICK * 128:(c + 1) * ICK * 128], cr[...],
            preferred_element_type=jnp.float32)
    acc = acct.T + jnp.dot(sf, be2_ref[...], preferred_element_type=jnp.float32)
    out_ref[:, 0:DIM] = acc
    out_ref[:, DIM:PADW] = jnp.ones((MBLK, PADW - DIM), jnp.float32)


def _gru_body(out_ref, aga_ref, agb_ref, wroot_ref, bconv_ref, wi_ref, wh_ref,
              bi_ref, bh_ref, new_ref):
    h = out_ref[:, 0:DIM]
    ag = aga_ref[0] + aga_ref[1] + agb_ref[0] + agb_ref[1]
    deg = jnp.maximum(ag[:, DIM:DIM + 1], 1.0)
    aggr = ag[:, 0:DIM] / deg
    m = jax.nn.relu(
        jnp.dot(h, wroot_ref[...], preferred_element_type=jnp.float32)
        + aggr + bconv_ref[...])
    gi = jnp.dot(m, wi_ref[...], preferred_element_type=jnp.float32) + bi_ref[...]
    gh = jnp.dot(h, wh_ref[...], preferred_element_type=jnp.float32) + bh_ref[...]
    r = _sigmoid(gi[:, 0:DIM] + gh[:, 0:DIM])
    z = _sigmoid(gi[:, DIM:2 * DIM] + gh[:, DIM:2 * DIM])
    n = jnp.tanh(gi[:, 2 * DIM:3 * DIM] + r * gh[:, 2 * DIM:3 * DIM])
    new_ref[:, 0:DIM] = (1.0 - z) * n + z * h
    new_ref[:, DIM:PADW] = jnp.zeros((N, PADW - DIM), jnp.float32)


def _s2s_body(out_ref, bcol_ref, brow_ref, wi_ref, wh_ref, bi_ref, bh_ref,
              w1_ref, b1_ref, w2_ref, b2_ref, o_ref):
    xo = out_ref[:, 0:DIM]
    bcol = bcol_ref[...]
    gid = lax.broadcasted_iota(jnp.int32, (N, B), 1)
    ohb = (gid == bcol).astype(jnp.float32)
    gid_t = lax.broadcasted_iota(jnp.int32, (B, N), 0)
    brow = brow_ref[0:1, :]
    ohb_t = (gid_t == brow).astype(jnp.float32)

    q_star = jnp.zeros((B, 2 * DIM), jnp.float32)
    hs = jnp.zeros((B, DIM), jnp.float32)
    cs = jnp.zeros((B, DIM), jnp.float32)
    for _ in range(3):
        gates = (jnp.dot(q_star, wi_ref[...], preferred_element_type=jnp.float32)
                 + bi_ref[...]
                 + jnp.dot(hs, wh_ref[...], preferred_element_type=jnp.float32)
                 + bh_ref[...])
        ig = _sigmoid(gates[:, 0:DIM])
        fg = _sigmoid(gates[:, DIM:2 * DIM])
        gg = jnp.tanh(gates[:, 2 * DIM:3 * DIM])
        og = _sigmoid(gates[:, 3 * DIM:4 * DIM])
        cs = fg * cs + ig * gg
        hs = og * jnp.tanh(cs)
        q_per = jnp.dot(ohb, hs, preferred_element_type=jnp.float32)
        e = jnp.sum(xo * q_per, axis=1, keepdims=True)
        masked = jnp.where(ohb > 0.5, e, -1e38)
        seg_max = jnp.max(masked, axis=0, keepdims=True)
        seg_max = jnp.where(seg_max < -1e30, 0.0, seg_max)
        gmax = jnp.sum(ohb * seg_max, axis=1, keepdims=True)
        exp_e = jnp.exp(e - gmax)
        denom = jnp.sum(ohb * exp_e, axis=0, keepdims=True)
        gden = jnp.sum(ohb * denom, axis=1, keepdims=True)
        a = exp_e / gden
        r_read = jnp.dot(ohb_t, a * xo, preferred_element_type=jnp.float32)
        q_star = jnp.concatenate([hs, r_read], axis=1)
    hid = jax.nn.relu(
        jnp.dot(q_star, w1_ref[...], preferred_element_type=jnp.float32)
        + b1_ref[...])
    o_ref[...] = (jnp.dot(hid, w2_ref[...], preferred_element_type=jnp.float32)
                  + b2_ref[...])


@functools.lru_cache(maxsize=None)
def _sc_kernels(ne):
    mesh = plsc.VectorSubcoreMesh(core_axis_name="c", subcore_axis_name="s")
    epw = ne // NW
    nch = epw // CHUNK

    @functools.partial(
        pl.kernel, mesh=mesh,
        out_type=jax.ShapeDtypeStruct((ne, PADW), jnp.float32),
        scratch_types=[
            pltpu.VMEM((nch, CHUNK), jnp.int32),
            pltpu.VMEM((epw, PADW), jnp.float32),
            pltpu.SemaphoreType.DMA,
        ])
    def sc_gather(table_hbm, idx_hbm, out_hbm, idx_v, rows_v, sem):
        wid = lax.axis_index("s") * 2 + lax.axis_index("c")
        base = wid * epw
        pltpu.sync_copy(idx_hbm.at[wid], idx_v)
        cps = [pltpu.async_copy(table_hbm.at[idx_v.at[j]],
                                rows_v.at[pl.ds(j * CHUNK, CHUNK)], sem)
               for j in range(nch)]
        for cp in cps:
            cp.wait()
        pltpu.sync_copy(rows_v, out_hbm.at[pl.ds(base, epw)])

    @functools.partial(
        pl.kernel, mesh=mesh,
        out_type=jax.ShapeDtypeStruct((2, N, PADW), jnp.float32),
        scratch_types=[
            pltpu.VMEM((nch, CHUNK), jnp.int32),
            pltpu.VMEM((epw, PADW), jnp.float32),
            pltpu.VMEM_SHARED((N, PADW), jnp.float32),
            pltpu.SemaphoreType.DMA,
        ])
    def sc_scatter(msg_hbm, idx_hbm, zeros_hbm, out_hbm,
                   idx_v, rows_v, shared, sem):
        cid = lax.axis_index("c")
        sid = lax.axis_index("s")
        wid = sid * 2 + cid
        base = wid * epw
        pltpu.sync_copy(zeros_hbm.at[pl.ds(sid * RPW, RPW)],
                        shared.at[pl.ds(sid * RPW, RPW)])
        pltpu.sync_copy(idx_hbm.at[wid], idx_v)
        pltpu.sync_copy(msg_hbm.at[pl.ds(base, epw)], rows_v)
        plsc.subcore_barrier()
        for j in range(nch):
            pltpu.sync_copy(rows_v.at[pl.ds(j * CHUNK, CHUNK)],
                            shared.at[idx_v.at[j]], add=True)
        plsc.subcore_barrier()
        pltpu.sync_copy(shared.at[pl.ds(sid * RPW, RPW)],
                        out_hbm.at[cid, pl.ds(sid * RPW, RPW)])

    return sc_gather, sc_scatter


_prologue = pl.pallas_call(
    _prologue_body,
    out_shape=(jax.ShapeDtypeStruct((N, PADW), jnp.float32),
               jax.ShapeDtypeStruct((2 * DIM, E), jnp.bfloat16)))

@functools.lru_cache(maxsize=None)
def _msg_call(ne, off):
    ob = off // MBLK
    return pl.pallas_call(
        _msg_body,
        grid=(ne // MBLK,),
        in_specs=[
            pl.BlockSpec((2 * DIM, MBLK), lambda i: (0, i + ob)),
            pl.BlockSpec((MBLK, PADW), lambda i: (i, 0)),
            pl.BlockSpec((DIM, DIM * 2 * DIM), lambda i: (0, 0)),
            pl.BlockSpec((DIM, DIM), lambda i: (0, 0)),
        ],
        out_specs=pl.BlockSpec((MBLK, PADW), lambda i: (i, 0)),
        out_shape=jax.ShapeDtypeStruct((ne, PADW), jnp.float32),
        scratch_shapes=[pltpu.VMEM((ICK * 2 * DIM, MBLK), jnp.bfloat16)
                        for _ in range(NCK)])

_gru = pl.pallas_call(
    _gru_body,
    out_shape=jax.ShapeDtypeStruct((N, PADW), jnp.float32))

_s2s = pl.pallas_call(
    _s2s_body,
    out_shape=jax.ShapeDtypeStruct((B, OUT_CLASSES), jnp.float32))


def kernel(x, edge_index, edge_attr, batch, W0, b0, We1, be1, We2, be2,
           Wroot, bconv, gru_Wi, gru_Wh, gru_bi, gru_bh,
           lstm_Wi, lstm_Wh, lstm_bi, lstm_bh, W1, b1, W2, b2):
    hne = E // 2
    src = edge_index[0].reshape(2, NW, hne // NW // CHUNK, CHUNK)
    dst = edge_index[1].reshape(2, NW, hne // NW // CHUNK, CHUNK)
    zeros = jnp.zeros((N, PADW), jnp.float32)
    we2b = (We2.reshape(2 * DIM, DIM, DIM).transpose(2, 1, 0)
            .reshape(DIM, DIM * 2 * DIM).astype(jnp.bfloat16))
    be2r = be2.reshape(DIM, DIM)

    sc_gather, sc_scatter = _sc_kernels(hne)
    msg_a = _msg_call(hne, 0)
    msg_b = _msg_call(hne, hne)
    out, eht = _prologue(x, edge_attr.T, W0, b0.reshape(1, DIM),
                         We1.T, be1.reshape(2 * DIM, 1))
    for _ in range(3):
        sfa = sc_gather(out, src[0])
        sfb = sc_gather(out, src[1])
        msga = msg_a(eht, sfa, we2b, be2r)
        agpa = sc_scatter(msga, dst[0], zeros)
        msgb = msg_b(eht, sfb, we2b, be2r)
        agpb = sc_scatter(msgb, dst[1], zeros)
        out = _gru(out, agpa, agpb, Wroot, bconv.reshape(1, DIM),
                   gru_Wi, gru_Wh, gru_bi.reshape(1, 3 * DIM),
                   gru_bh.reshape(1, 3 * DIM))
    bcol = batch.reshape(N, 1)
    brow = jnp.broadcast_to(batch[None, :], (8, N))
    return _s2s(out, bcol, brow, lstm_Wi, lstm_Wh,
                lstm_bi.reshape(1, 4 * DIM), lstm_bh.reshape(1, 4 * DIM),
                W1, b1.reshape(1, DIM), W2, b2.reshape(1, OUT_CLASSES))


# speedup vs baseline: 1.0018x; 1.0018x over previous
import functools

import jax
import jax.numpy as jnp
from jax import lax
from jax.experimental import pallas as pl
from jax.experimental.pallas import tpu as pltpu
from jax.experimental.pallas import tpu_sc as plsc

N = 4096
E = 16384
F_IN = 11
DIM = 64
EDGE_DIM = 6
B = 256
OUT_CLASSES = 12

NW = 32
EPW = E // NW
CHUNK = 128
NCH = EPW // CHUNK
PADW = 128
RPW = N // 16


def _sigmoid(v):
    return 1.0 / (1.0 + jnp.exp(-v))


def _prologue_body(x_ref, eat_ref, w0_ref, b0_ref, we1t_ref, be1_ref,
                   out_ref, eht_ref):
    out_ref[:, 0:DIM] = jax.nn.relu(
        jnp.dot(x_ref[...], w0_ref[...], preferred_element_type=jnp.float32)
        + b0_ref[...])
    out_ref[:, DIM:PADW] = jnp.zeros((N, PADW - DIM), jnp.float32)
    eht = jax.nn.relu(
        jnp.dot(we1t_ref[...], eat_ref[...], preferred_element_type=jnp.float32)
        + be1_ref[...])
    eht_ref[...] = eht.astype(jnp.bfloat16)


MBLK = 1024
NCK = 4
ICK = DIM // NCK


def _msg_body(eht_ref, sf_ref, we2t_ref, be2_ref, out_ref, *c_refs):
    sf = sf_ref[:, 0:DIM]
    sft = sf.T.astype(jnp.bfloat16)
    eht = eht_ref[...]
    acct = jnp.zeros((DIM, MBLK), jnp.float32)
    for c in range(NCK):
        cr = c_refs[c]
        for k in range(ICK):
            i = c * ICK + k
            cr[k * 128:(k + 1) * 128, :] = eht * sft[i:i + 1, :]
        acct = acct + jnp.dot(
            we2t_ref[:, c * ICK * 128:(c + 1) * ICK * 128], cr[...],
            preferred_element_type=jnp.float32)
    acc = acct.T + jnp.dot(sf, be2_ref[...], preferred_element_type=jnp.float32)
    out_ref[:, 0:DIM] = acc
    out_ref[:, DIM:PADW] = jnp.ones((MBLK, PADW - DIM), jnp.float32)


def _gru_body(out_ref, ag_ref, wroot_ref, bconv_ref, wi_ref, wh_ref,
              bi_ref, bh_ref, new_ref):
    h = out_ref[:, 0:DIM]
    ag = ag_ref[0] + ag_ref[1]
    deg = jnp.maximum(ag[:, DIM:DIM + 1], 1.0)
    aggr = ag[:, 0:DIM] / deg
    m = jax.nn.relu(
        jnp.dot(h, wroot_ref[...], preferred_element_type=jnp.float32)
        + aggr + bconv_ref[...])
    gi = jnp.dot(m, wi_ref[...], preferred_element_type=jnp.float32) + bi_ref[...]
    gh = jnp.dot(h, wh_ref[...], preferred_element_type=jnp.float32) + bh_ref[...]
    r = _sigmoid(gi[:, 0:DIM] + gh[:, 0:DIM])
    z = _sigmoid(gi[:, DIM:2 * DIM] + gh[:, DIM:2 * DIM])
    n = jnp.tanh(gi[:, 2 * DIM:3 * DIM] + r * gh[:, 2 * DIM:3 * DIM])
    new_ref[:, 0:DIM] = (1.0 - z) * n + z * h
    new_ref[:, DIM:PADW] = jnp.zeros((N, PADW - DIM), jnp.float32)


def _s2s_body(out_ref, bcol_ref, brow_ref, wi_ref, wh_ref, bi_ref, bh_ref,
              w1_ref, b1_ref, w2_ref, b2_ref, o_ref):
    xo = out_ref[:, 0:DIM]
    bcol = bcol_ref[...]
    gid = lax.broadcasted_iota(jnp.int32, (N, B), 1)
    ohb = (gid == bcol).astype(jnp.float32)
    gid_t = lax.broadcasted_iota(jnp.int32, (B, N), 0)
    brow = brow_ref[0:1, :]
    ohb_t = (gid_t == brow).astype(jnp.float32)

    q_star = jnp.zeros((B, 2 * DIM), jnp.float32)
    hs = jnp.zeros((B, DIM), jnp.float32)
    cs = jnp.zeros((B, DIM), jnp.float32)
    for _ in range(3):
        gates = (jnp.dot(q_star, wi_ref[...], preferred_element_type=jnp.float32)
                 + bi_ref[...]
                 + jnp.dot(hs, wh_ref[...], preferred_element_type=jnp.float32)
                 + bh_ref[...])
        ig = _sigmoid(gates[:, 0:DIM])
        fg = _sigmoid(gates[:, DIM:2 * DIM])
        gg = jnp.tanh(gates[:, 2 * DIM:3 * DIM])
        og = _sigmoid(gates[:, 3 * DIM:4 * DIM])
        cs = fg * cs + ig * gg
        hs = og * jnp.tanh(cs)
        q_per = jnp.dot(ohb, hs, preferred_element_type=jnp.float32)
        e = jnp.sum(xo * q_per, axis=1, keepdims=True)
        masked = jnp.where(ohb > 0.5, e, -1e38)
        seg_max = jnp.max(masked, axis=0, keepdims=True)
        seg_max = jnp.where(seg_max < -1e30, 0.0, seg_max)
        gmax = jnp.sum(ohb * seg_max, axis=1, keepdims=True)
        exp_e = jnp.exp(e - gmax)
        denom = jnp.sum(ohb * exp_e, axis=0, keepdims=True)
        gden = jnp.sum(ohb * denom, axis=1, keepdims=True)
        a = exp_e / gden
        r_read = jnp.dot(ohb_t, a * xo, preferred_element_type=jnp.float32)
        q_star = jnp.concatenate([hs, r_read], axis=1)
    hid = jax.nn.relu(
        jnp.dot(q_star, w1_ref[...], preferred_element_type=jnp.float32)
        + b1_ref[...])
    o_ref[...] = (jnp.dot(hid, w2_ref[...], preferred_element_type=jnp.float32)
                  + b2_ref[...])


@functools.lru_cache(maxsize=None)
def _sc_kernels(ne):
    mesh = plsc.VectorSubcoreMesh(core_axis_name="c", subcore_axis_name="s")
    epw = ne // NW
    nch = epw // CHUNK

    @functools.partial(
        pl.kernel, mesh=mesh,
        out_type=jax.ShapeDtypeStruct((ne, PADW), jnp.float32),
        scratch_types=[
            pltpu.VMEM((nch, CHUNK), jnp.int32),
            pltpu.VMEM((epw, PADW), jnp.float32),
            pltpu.SemaphoreType.DMA,
        ])
    def sc_gather(table_hbm, idx_hbm, out_hbm, idx_v, rows_v, sem):
        wid = lax.axis_index("s") * 2 + lax.axis_index("c")
        base = wid * epw
        pltpu.sync_copy(idx_hbm.at[wid], idx_v)
        cps = [pltpu.async_copy(table_hbm.at[idx_v.at[j]],
                                rows_v.at[pl.ds(j * CHUNK, CHUNK)], sem)
               for j in range(nch)]
        for cp in cps:
            cp.wait()
        pltpu.sync_copy(rows_v, out_hbm.at[pl.ds(base, epw)])

    @functools.partial(
        pl.kernel, mesh=mesh,
        out_type=jax.ShapeDtypeStruct((2, N, PADW), jnp.float32),
        scratch_types=[
            pltpu.VMEM((nch, CHUNK), jnp.int32),
            pltpu.VMEM((epw, PADW), jnp.float32),
            pltpu.VMEM_SHARED((N, PADW), jnp.float32),
            pltpu.SemaphoreType.DMA,
        ])
    def sc_scatter(msg_hbm, idx_hbm, zeros_hbm, out_hbm,
                   idx_v, rows_v, shared, sem):
        cid = lax.axis_index("c")
        sid = lax.axis_index("s")
        wid = sid * 2 + cid
        base = wid * epw
        pltpu.sync_copy(zeros_hbm.at[pl.ds(sid * RPW, RPW)],
                        shared.at[pl.ds(sid * RPW, RPW)])
        pltpu.sync_copy(idx_hbm.at[wid], idx_v)
        pltpu.sync_copy(msg_hbm.at[pl.ds(base, epw)], rows_v)
        plsc.subcore_barrier()
        for j in range(nch):
            pltpu.sync_copy(rows_v.at[pl.ds(j * CHUNK, CHUNK)],
                            shared.at[idx_v.at[j]], add=True)
        plsc.subcore_barrier()
        pltpu.sync_copy(shared.at[pl.ds(sid * RPW, RPW)],
                        out_hbm.at[cid, pl.ds(sid * RPW, RPW)])

    return sc_gather, sc_scatter


_prologue = pl.pallas_call(
    _prologue_body,
    out_shape=(jax.ShapeDtypeStruct((N, PADW), jnp.float32),
               jax.ShapeDtypeStruct((2 * DIM, E), jnp.bfloat16)))

@functools.lru_cache(maxsize=None)
def _msg_call(ne, off):
    ob = off // MBLK
    return pl.pallas_call(
        _msg_body,
        grid=(ne // MBLK,),
        in_specs=[
            pl.BlockSpec((2 * DIM, MBLK), lambda i: (0, i + ob)),
            pl.BlockSpec((MBLK, PADW), lambda i: (i, 0)),
            pl.BlockSpec((DIM, DIM * 2 * DIM), lambda i: (0, 0)),
            pl.BlockSpec((DIM, DIM), lambda i: (0, 0)),
        ],
        out_specs=pl.BlockSpec((MBLK, PADW), lambda i: (i, 0)),
        out_shape=jax.ShapeDtypeStruct((ne, PADW), jnp.float32),
        scratch_shapes=[pltpu.VMEM((ICK * 2 * DIM, MBLK), jnp.bfloat16)
                        for _ in range(NCK)])

_gru = pl.pallas_call(
    _gru_body,
    out_shape=jax.ShapeDtypeStruct((N, PADW), jnp.float32))

_s2s = pl.pallas_call(
    _s2s_body,
    out_shape=jax.ShapeDtypeStruct((B, OUT_CLASSES), jnp.float32))


def kernel(x, edge_index, edge_attr, batch, W0, b0, We1, be1, We2, be2,
           Wroot, bconv, gru_Wi, gru_Wh, gru_bi, gru_bh,
           lstm_Wi, lstm_Wh, lstm_bi, lstm_bh, W1, b1, W2, b2):
    src = edge_index[0].reshape(NW, NCH, CHUNK)
    dst = edge_index[1].reshape(NW, NCH, CHUNK)
    zeros = jnp.zeros((N, PADW), jnp.float32)
    we2b = (We2.reshape(2 * DIM, DIM, DIM).transpose(2, 1, 0)
            .reshape(DIM, DIM * 2 * DIM).astype(jnp.bfloat16))
    be2r = be2.reshape(DIM, DIM)

    sc_gather, sc_scatter = _sc_kernels(E)
    msg = _msg_call(E, 0)
    out, eht = _prologue(x, edge_attr.T, W0, b0.reshape(1, DIM),
                         We1.T, be1.reshape(2 * DIM, 1))
    for _ in range(3):
        sf = sc_gather(out, src)
        msgv = msg(eht, sf, we2b, be2r)
        agp = sc_scatter(msgv, dst, zeros)
        out = _gru(out, agp, Wroot, bconv.reshape(1, DIM),
                   gru_Wi, gru_Wh, gru_bi.reshape(1, 3 * DIM),
                   gru_bh.reshape(1, 3 * DIM))
    bcol = batch.reshape(N, 1)
    brow = jnp.broadcast_to(batch[None, :], (8, N))
    return _s2s(out, bcol, brow, lstm_Wi, lstm_Wh,
                lstm_bi.reshape(1, 4 * DIM), lstm_bh.reshape(1, 4 * DIM),
                W1, b1.reshape(1, DIM), W2, b2.reshape(1, OUT_CLASSES))


# speedup vs baseline: 1.0019x; 1.0001x over previous
import functools

import jax
import jax.numpy as jnp
from jax import lax
from jax.experimental import pallas as pl
from jax.experimental.pallas import tpu as pltpu
from jax.experimental.pallas import tpu_sc as plsc

N = 4096
E = 16384
F_IN = 11
DIM = 64
EDGE_DIM = 6
B = 256
OUT_CLASSES = 12

NW = 32
EPW = E // NW
CHUNK = 128
NCH = EPW // CHUNK
PADW = 128
RPW = N // 16


def _sigmoid(v):
    return 1.0 / (1.0 + jnp.exp(-v))


def _prologue_body(x_ref, eat_ref, w0_ref, b0_ref, we1t_ref, be1_ref,
                   out_ref, eht_ref):
    out_ref[:, 0:DIM] = jax.nn.relu(
        jnp.dot(x_ref[...], w0_ref[...], preferred_element_type=jnp.float32)
        + b0_ref[...])
    out_ref[:, DIM:PADW] = jnp.zeros((N, PADW - DIM), jnp.float32)
    eht = jax.nn.relu(
        jnp.dot(we1t_ref[...], eat_ref[...], preferred_element_type=jnp.float32)
        + be1_ref[...])
    eht_ref[...] = eht.astype(jnp.bfloat16)


MBLK = 1024
NCK = 8
ICK = DIM // NCK


def _msg_body(eht_ref, sf_ref, we2t_ref, be2_ref, out_ref, *c_refs):
    sf = sf_ref[:, 0:DIM]
    sft = sf.T.astype(jnp.bfloat16)
    eht = eht_ref[...]
    acct = jnp.zeros((DIM, MBLK), jnp.float32)
    for c in range(NCK):
        cr = c_refs[c]
        for k in range(ICK):
            i = c * ICK + k
            cr[k * 128:(k + 1) * 128, :] = eht * sft[i:i + 1, :]
        acct = acct + jnp.dot(
            we2t_ref[:, c * ICK * 128:(c + 1) * ICK * 128], cr[...],
            preferred_element_type=jnp.float32)
    acc = acct.T + jnp.dot(sf, be2_ref[...], preferred_element_type=jnp.float32)
    out_ref[:, 0:DIM] = acc
    out_ref[:, DIM:PADW] = jnp.ones((MBLK, PADW - DIM), jnp.float32)


def _gru_body(out_ref, ag_ref, wroot_ref, bconv_ref, wi_ref, wh_ref,
              bi_ref, bh_ref, new_ref):
    h = out_ref[:, 0:DIM]
    ag = ag_ref[0] + ag_ref[1]
    deg = jnp.maximum(ag[:, DIM:DIM + 1], 1.0)
    aggr = ag[:, 0:DIM] / deg
    m = jax.nn.relu(
        jnp.dot(h, wroot_ref[...], preferred_element_type=jnp.float32)
        + aggr + bconv_ref[...])
    gi = jnp.dot(m, wi_ref[...], preferred_element_type=jnp.float32) + bi_ref[...]
    gh = jnp.dot(h, wh_ref[...], preferred_element_type=jnp.float32) + bh_ref[...]
    r = _sigmoid(gi[:, 0:DIM] + gh[:, 0:DIM])
    z = _sigmoid(gi[:, DIM:2 * DIM] + gh[:, DIM:2 * DIM])
    n = jnp.tanh(gi[:, 2 * DIM:3 * DIM] + r * gh[:, 2 * DIM:3 * DIM])
    new_ref[:, 0:DIM] = (1.0 - z) * n + z * h
    new_ref[:, DIM:PADW] = jnp.zeros((N, PADW - DIM), jnp.float32)


def _s2s_body(out_ref, bcol_ref, brow_ref, wi_ref, wh_ref, bi_ref, bh_ref,
              w1_ref, b1_ref, w2_ref, b2_ref, o_ref):
    xo = out_ref[:, 0:DIM]
    bcol = bcol_ref[...]
    gid = lax.broadcasted_iota(jnp.int32, (N, B), 1)
    ohb = (gid == bcol).astype(jnp.float32)
    gid_t = lax.broadcasted_iota(jnp.int32, (B, N), 0)
    brow = brow_ref[0:1, :]
    ohb_t = (gid_t == brow).astype(jnp.float32)

    q_star = jnp.zeros((B, 2 * DIM), jnp.float32)
    hs = jnp.zeros((B, DIM), jnp.float32)
    cs = jnp.zeros((B, DIM), jnp.float32)
    for _ in range(3):
        gates = (jnp.dot(q_star, wi_ref[...], preferred_element_type=jnp.float32)
                 + bi_ref[...]
                 + jnp.dot(hs, wh_ref[...], preferred_element_type=jnp.float32)
                 + bh_ref[...])
        ig = _sigmoid(gates[:, 0:DIM])
        fg = _sigmoid(gates[:, DIM:2 * DIM])
        gg = jnp.tanh(gates[:, 2 * DIM:3 * DIM])
        og = _sigmoid(gates[:, 3 * DIM:4 * DIM])
        cs = fg * cs + ig * gg
        hs = og * jnp.tanh(cs)
        q_per = jnp.dot(ohb, hs, preferred_element_type=jnp.float32)
        e = jnp.sum(xo * q_per, axis=1, keepdims=True)
        masked = jnp.where(ohb > 0.5, e, -1e38)
        seg_max = jnp.max(masked, axis=0, keepdims=True)
        seg_max = jnp.where(seg_max < -1e30, 0.0, seg_max)
        gmax = jnp.sum(ohb * seg_max, axis=1, keepdims=True)
        exp_e = jnp.exp(e - gmax)
        denom = jnp.sum(ohb * exp_e, axis=0, keepdims=True)
        gden = jnp.sum(ohb * denom, axis=1, keepdims=True)
        a = exp_e / gden
        r_read = jnp.dot(ohb_t, a * xo, preferred_element_type=jnp.float32)
        q_star = jnp.concatenate([hs, r_read], axis=1)
    hid = jax.nn.relu(
        jnp.dot(q_star, w1_ref[...], preferred_element_type=jnp.float32)
        + b1_ref[...])
    o_ref[...] = (jnp.dot(hid, w2_ref[...], preferred_element_type=jnp.float32)
                  + b2_ref[...])


@functools.lru_cache(maxsize=None)
def _sc_kernels(ne):
    mesh = plsc.VectorSubcoreMesh(core_axis_name="c", subcore_axis_name="s")
    epw = ne // NW
    nch = epw // CHUNK

    @functools.partial(
        pl.kernel, mesh=mesh,
        out_type=jax.ShapeDtypeStruct((ne, PADW), jnp.float32),
        scratch_types=[
            pltpu.VMEM((nch, CHUNK), jnp.int32),
            pltpu.VMEM((epw, PADW), jnp.float32),
            pltpu.SemaphoreType.DMA,
        ])
    def sc_gather(table_hbm, idx_hbm, out_hbm, idx_v, rows_v, sem):
        wid = lax.axis_index("s") * 2 + lax.axis_index("c")
        base = wid * epw
        pltpu.sync_copy(idx_hbm.at[wid], idx_v)
        cps = [pltpu.async_copy(table_hbm.at[idx_v.at[j]],
                                rows_v.at[pl.ds(j * CHUNK, CHUNK)], sem)
               for j in range(nch)]
        for cp in cps:
            cp.wait()
        pltpu.sync_copy(rows_v, out_hbm.at[pl.ds(base, epw)])

    @functools.partial(
        pl.kernel, mesh=mesh,
        out_type=jax.ShapeDtypeStruct((2, N, PADW), jnp.float32),
        scratch_types=[
            pltpu.VMEM((nch, CHUNK), jnp.int32),
            pltpu.VMEM((epw, PADW), jnp.float32),
            pltpu.VMEM_SHARED((N, PADW), jnp.float32),
            pltpu.SemaphoreType.DMA,
        ])
    def sc_scatter(msg_hbm, idx_hbm, zeros_hbm, out_hbm,
                   idx_v, rows_v, shared, sem):
        cid = lax.axis_index("c")
        sid = lax.axis_index("s")
        wid = sid * 2 + cid
        base = wid * epw
        pltpu.sync_copy(zeros_hbm.at[pl.ds(sid * RPW, RPW)],
                        shared.at[pl.ds(sid * RPW, RPW)])
        pltpu.sync_copy(idx_hbm.at[wid], idx_v)
        pltpu.sync_copy(msg_hbm.at[pl.ds(base, epw)], rows_v)
        plsc.subcore_barrier()
        for j in range(nch):
            pltpu.sync_copy(rows_v.at[pl.ds(j * CHUNK, CHUNK)],
                            shared.at[idx_v.at[j]], add=True)
        plsc.subcore_barrier()
        pltpu.sync_copy(shared.at[pl.ds(sid * RPW, RPW)],
                        out_hbm.at[cid, pl.ds(sid * RPW, RPW)])

    return sc_gather, sc_scatter


_prologue = pl.pallas_call(
    _prologue_body,
    out_shape=(jax.ShapeDtypeStruct((N, PADW), jnp.float32),
               jax.ShapeDtypeStruct((2 * DIM, E), jnp.bfloat16)))

@functools.lru_cache(maxsize=None)
def _msg_call(ne, off):
    ob = off // MBLK
    return pl.pallas_call(
        _msg_body,
        grid=(ne // MBLK,),
        in_specs=[
            pl.BlockSpec((2 * DIM, MBLK), lambda i: (0, i + ob)),
            pl.BlockSpec((MBLK, PADW), lambda i: (i, 0)),
            pl.BlockSpec((DIM, DIM * 2 * DIM), lambda i: (0, 0)),
            pl.BlockSpec((DIM, DIM), lambda i: (0, 0)),
        ],
        out_specs=pl.BlockSpec((MBLK, PADW), lambda i: (i, 0)),
        out_shape=jax.ShapeDtypeStruct((ne, PADW), jnp.float32),
        scratch_shapes=[pltpu.VMEM((ICK * 2 * DIM, MBLK), jnp.bfloat16)
                        for _ in range(NCK)])

_gru = pl.pallas_call(
    _gru_body,
    out_shape=jax.ShapeDtypeStruct((N, PADW), jnp.float32))

_s2s = pl.pallas_call(
    _s2s_body,
    out_shape=jax.ShapeDtypeStruct((B, OUT_CLASSES), jnp.float32))


def kernel(x, edge_index, edge_attr, batch, W0, b0, We1, be1, We2, be2,
           Wroot, bconv, gru_Wi, gru_Wh, gru_bi, gru_bh,
           lstm_Wi, lstm_Wh, lstm_bi, lstm_bh, W1, b1, W2, b2):
    src = edge_index[0].reshape(NW, NCH, CHUNK)
    dst = edge_index[1].reshape(NW, NCH, CHUNK)
    zeros = jnp.zeros((N, PADW), jnp.float32)
    we2b = (We2.reshape(2 * DIM, DIM, DIM).transpose(2, 1, 0)
            .reshape(DIM, DIM * 2 * DIM).astype(jnp.bfloat16))
    be2r = be2.reshape(DIM, DIM)

    sc_gather, sc_scatter = _sc_kernels(E)
    msg = _msg_call(E, 0)
    out, eht = _prologue(x, edge_attr.T, W0, b0.reshape(1, DIM),
                         We1.T, be1.reshape(2 * DIM, 1))
    for _ in range(3):
        sf = sc_gather(out, src)
        msgv = msg(eht, sf, we2b, be2r)
        agp = sc_scatter(msgv, dst, zeros)
        out = _gru(out, agp, Wroot, bconv.reshape(1, DIM),
                   gru_Wi, gru_Wh, gru_bi.reshape(1, 3 * DIM),
                   gru_bh.reshape(1, 3 * DIM))
    bcol = batch.reshape(N, 1)
    brow = jnp.broadcast_to(batch[None, :], (8, N))
    return _s2s(out, bcol, brow, lstm_Wi, lstm_Wh,
                lstm_bi.reshape(1, 4 * DIM), lstm_bh.reshape(1, 4 * DIM),
                W1, b1.reshape(1, DIM), W2, b2.reshape(1, OUT_CLASSES))
